# baseline (speedup 1.0000x reference)
import jax
import jax.numpy as jnp
from jax.experimental import pallas as pl
from jax.experimental.pallas import tpu as pltpu

_BLK = 2048


def _head_block_kernel(env_ref, h_ref, w_ref, b_ref, out_ref, *, n_env, a_dim):
    h_bf = h_ref[...].astype(jnp.bfloat16)
    y = jnp.dot(h_bf, w_ref[...], preferred_element_type=jnp.float32)
    y = y + b_ref[...]
    env = env_ref[0]
    out = jnp.zeros((y.shape[0], a_dim), jnp.float32)
    for e in range(n_env):
        out = jnp.where(env == e, y[:, e * a_dim:(e + 1) * a_dim], out)
    out_ref[...] = out


def kernel(h, env_ids, W, b):
    n, d = h.shape
    n_env, _, a_dim = W.shape
    blk = _BLK
    grid = n // blk

    env3 = env_ids.reshape(-1).astype(jnp.int32).reshape(grid, blk, 1)
    w_flat = W.transpose(1, 0, 2).reshape(d, n_env * a_dim).astype(jnp.bfloat16)
    b_flat = b.reshape(1, n_env * a_dim)

    import functools
    body = functools.partial(_head_block_kernel, n_env=n_env, a_dim=a_dim)
    out = pl.pallas_call(
        body,
        grid=(grid,),
        in_specs=[
            pl.BlockSpec((1, blk, 1), lambda i: (i, 0, 0)),
            pl.BlockSpec((blk, d), lambda i: (i, 0)),
            pl.BlockSpec((d, n_env * a_dim), lambda i: (0, 0)),
            pl.BlockSpec((1, n_env * a_dim), lambda i: (0, 0)),
        ],
        out_specs=pl.BlockSpec((blk, a_dim), lambda i: (i, 0)),
        out_shape=jax.ShapeDtypeStruct((n, a_dim), jnp.float32),
        compiler_params=pltpu.CompilerParams(
            dimension_semantics=("parallel",),
        ),
    )(env3, h, w_flat, b_flat)
    return out


# speedup vs baseline: 1.2672x; 1.2672x over previous
import jax
import jax.numpy as jnp
from jax.experimental import pallas as pl
from jax.experimental.pallas import tpu as pltpu

_BLK = 2048


def _head_block_kernel(env_ref, h_ref, w_ref, b_ref, out_ref, *, n_env, a_dim):
    ea = n_env * a_dim
    h_bf = h_ref[...].astype(jnp.bfloat16)
    y = jnp.dot(h_bf, w_ref[...], preferred_element_type=jnp.float32)
    y = y + b_ref[...]
    env = env_ref[0]
    col_env = jax.lax.broadcasted_iota(jnp.int32, (1, ea), 1) // a_dim
    masked = jnp.where(env == col_env, y, 0.0)
    c_mod = jax.lax.broadcasted_iota(jnp.int32, (ea, a_dim), 0) % a_dim
    j_col = jax.lax.broadcasted_iota(jnp.int32, (ea, a_dim), 1)
    s_fold = (c_mod == j_col).astype(jnp.float32)
    out_ref[...] = jnp.dot(masked, s_fold, preferred_element_type=jnp.float32)


def kernel(h, env_ids, W, b):
    n, d = h.shape
    n_env, _, a_dim = W.shape
    blk = _BLK
    grid = n // blk

    env3 = env_ids.reshape(-1).astype(jnp.int32).reshape(grid, blk, 1)
    w_flat = W.transpose(1, 0, 2).reshape(d, n_env * a_dim).astype(jnp.bfloat16)
    b_flat = b.reshape(1, n_env * a_dim)

    import functools
    body = functools.partial(_head_block_kernel, n_env=n_env, a_dim=a_dim)
    out = pl.pallas_call(
        body,
        grid=(grid,),
        in_specs=[
            pl.BlockSpec((1, blk, 1), lambda i: (i, 0, 0)),
            pl.BlockSpec((blk, d), lambda i: (i, 0)),
            pl.BlockSpec((d, n_env * a_dim), lambda i: (0, 0)),
            pl.BlockSpec((1, n_env * a_dim), lambda i: (0, 0)),
        ],
        out_specs=pl.BlockSpec((blk, a_dim), lambda i: (i, 0)),
        out_shape=jax.ShapeDtypeStruct((n, a_dim), jnp.float32),
        compiler_params=pltpu.CompilerParams(
            dimension_semantics=("parallel",),
        ),
    )(env3, h, w_flat, b_flat)
    return out
